# baseline (speedup 1.0000x reference)
import functools

import jax
import jax.numpy as jnp
from jax import lax
from jax.experimental import pallas as pl
from jax.experimental.pallas import tpu as pltpu
from jax.experimental.pallas import tpu_sc as plsc

VOCAB = 1000000
EMB = 64
NROWS = 4096
SEQ = 200
NW = 32
XR_PER_W = NROWS // NW
P = 2
GROUPS = XR_PER_W // P
S0 = 128
S1 = SEQ - S0

_mesh = plsc.VectorSubcoreMesh(core_axis_name="c", subcore_axis_name="s")


@functools.partial(
    pl.kernel,
    mesh=_mesh,
    out_type=jax.ShapeDtypeStruct((NROWS, SEQ, EMB), jnp.float32),
    scratch_types=[
        pltpu.VMEM((XR_PER_W, SEQ), jnp.int32),
        pltpu.VMEM((2, P, SEQ, EMB), jnp.float32),
        pltpu.SemaphoreType.DMA,
        pltpu.SemaphoreType.DMA,
        pltpu.SemaphoreType.DMA,
        pltpu.SemaphoreType.DMA,
    ],
    compiler_params=pltpu.CompilerParams(use_tc_tiling_on_sc=False),
)
def _emb_lookup(x_hbm, tab_hbm, out_hbm, idx_v, rows_v, g0, g1, o0, o1):
    gsems = (g0, g1)
    osems = (o0, o1)
    wid = lax.axis_index("s") * 2 + lax.axis_index("c")
    xbase = wid * XR_PER_W
    pltpu.sync_copy(x_hbm.at[pl.ds(xbase, XR_PER_W)], idx_v)

    def fire(g, b):
        for p in range(P):
            row = idx_v.at[g * P + p]
            dst = rows_v.at[b].at[p]
            pltpu.async_copy(tab_hbm.at[row.at[pl.ds(0, S0)]],
                             dst.at[pl.ds(0, S0)], gsems[b])
            pltpu.async_copy(tab_hbm.at[row.at[pl.ds(S0, S1)]],
                             dst.at[pl.ds(S0, S1)], gsems[b])

    def wait_gather(b):
        pltpu.make_async_copy(out_hbm.at[pl.ds(0, P)], rows_v.at[b],
                              gsems[b]).wait()

    def start_write(g, b):
        pltpu.async_copy(rows_v.at[b], out_hbm.at[pl.ds(xbase + g * P, P)],
                         osems[b])

    def wait_write(b):
        pltpu.make_async_copy(rows_v.at[b], out_hbm.at[pl.ds(0, P)],
                              osems[b]).wait()

    fire(0, 0)
    fire(1, 1)
    wait_gather(0)
    start_write(0, 0)

    def body(h, carry):
        g_odd = 2 * h + 1
        wait_gather(1)
        start_write(g_odd, 1)
        wait_write(0)
        fire(g_odd + 1, 0)
        wait_gather(0)
        start_write(g_odd + 1, 0)
        wait_write(1)
        fire(g_odd + 2, 1)
        return carry

    lax.fori_loop(0, (GROUPS - 2) // 2, body, 0)

    wait_gather(1)
    start_write(GROUPS - 1, 1)
    wait_write(0)
    wait_write(1)


def kernel(x, emb_table):
    return _emb_lookup(x.astype(jnp.int32), emb_table)


# speedup vs baseline: 1.0047x; 1.0047x over previous
import functools

import jax
import jax.numpy as jnp
from jax import lax
from jax.experimental import pallas as pl
from jax.experimental.pallas import tpu as pltpu
from jax.experimental.pallas import tpu_sc as plsc

VOCAB = 1000000
EMB = 64
BATCH = 4096 * 200
NW = 32
ROWS_PER_W = BATCH // NW
C = 512
GROUPS = ROWS_PER_W // C

_mesh = plsc.VectorSubcoreMesh(core_axis_name="c", subcore_axis_name="s")


@functools.partial(
    pl.kernel,
    mesh=_mesh,
    out_type=jax.ShapeDtypeStruct((BATCH, EMB), jnp.float32),
    scratch_types=[
        pltpu.VMEM((ROWS_PER_W,), jnp.int32),
        pltpu.VMEM((2, C, EMB), jnp.float32),
        pltpu.SemaphoreType.DMA,
        pltpu.SemaphoreType.DMA,
        pltpu.SemaphoreType.DMA,
        pltpu.SemaphoreType.DMA,
    ],
    compiler_params=pltpu.CompilerParams(use_tc_tiling_on_sc=False),
)
def _emb_lookup(x_hbm, tab_hbm, out_hbm, idx_v, rows_v, g0, g1, o0, o1):
    gsems = (g0, g1)
    osems = (o0, o1)
    wid = lax.axis_index("s") * 2 + lax.axis_index("c")
    base = wid * ROWS_PER_W
    pltpu.sync_copy(x_hbm.at[pl.ds(base, ROWS_PER_W)], idx_v)

    def fire(g, b):
        pltpu.async_copy(tab_hbm.at[idx_v.at[pl.ds(g * C, C)]], rows_v.at[b],
                         gsems[b])

    def wait_gather(b):
        pltpu.make_async_copy(out_hbm.at[pl.ds(0, C)], rows_v.at[b],
                              gsems[b]).wait()

    def start_write(g, b):
        pltpu.async_copy(rows_v.at[b], out_hbm.at[pl.ds(base + g * C, C)],
                         osems[b])

    def wait_write(b):
        pltpu.make_async_copy(rows_v.at[b], out_hbm.at[pl.ds(0, C)],
                              osems[b]).wait()

    fire(0, 0)
    fire(1, 1)
    wait_gather(0)
    start_write(0, 0)

    def body(h, carry):
        g_odd = 2 * h + 1
        wait_gather(1)
        start_write(g_odd, 1)
        wait_write(0)
        fire(g_odd + 1, 0)
        wait_gather(0)
        start_write(g_odd + 1, 0)
        wait_write(1)
        fire(g_odd + 2, 1)
        return carry

    lax.fori_loop(0, (GROUPS - 2) // 2, body, 0)

    wait_gather(1)
    start_write(GROUPS - 1, 1)
    wait_write(0)
    wait_write(1)


def kernel(x, emb_table):
    xf = x.reshape(-1).astype(jnp.int32)
    out = _emb_lookup(xf, emb_table)
    return out.reshape(x.shape[0], x.shape[1], EMB)
